# baseline (speedup 1.0000x reference)
import functools

import jax
import jax.numpy as jnp
from jax import lax
from jax.experimental import pallas as pl
from jax.experimental.pallas import tpu as pltpu
from jax.experimental.pallas import tpu_sc as plsc

NC = 2
NS = 16
L = 16
CH = 128


def _rsqrt_nr(m):
    i = plsc.bitcast(m, jnp.int32)
    seed = jnp.full((L,), 0x5F3759DF, jnp.int32) - lax.shift_right_logical(i, 1)
    y = plsc.bitcast(seed, jnp.float32)
    for _ in range(3):
        y = y * (1.5 - 0.5 * m * y * y)
    return y


@functools.lru_cache(maxsize=None)
def _build(B, D, margin):
    NW = NC * NS
    half = B // 2
    P = half // NW
    NCH = (2 * P) // CH
    G = P // L

    mesh = plsc.VectorSubcoreMesh(
        core_axis_name="c", subcore_axis_name="s",
        num_cores=NC, num_subcores=NS)

    @functools.partial(
        pl.kernel,
        mesh=mesh,
        compiler_params=pltpu.CompilerParams(
            needs_layout_passes=False, use_tc_tiling_on_sc=False),
        out_type=(
            jax.ShapeDtypeStruct((half,), jnp.float32),
            jax.ShapeDtypeStruct((half,), jnp.float32),
            jax.ShapeDtypeStruct((NW, L), jnp.float32),
        ),
        scratch_types=[
            pltpu.VMEM((NCH, CH), jnp.int32),
            pltpu.VMEM((NCH, CH), jnp.int32),
            pltpu.VMEM((NCH, CH), jnp.int32),
            pltpu.VMEM((2 * P, D), jnp.float32),
            pltpu.VMEM((2 * P, D), jnp.float32),
            pltpu.VMEM((2 * P, D), jnp.float32),
            pltpu.VMEM((2 * P,), jnp.float32),
            pltpu.VMEM((L,), jnp.float32),
            pltpu.SemaphoreType.DMA,
        ],
    )
    def distmult(h_hbm, t_hbm, r_hbm, ent_hbm, rel_hbm,
                 pos_out, neg_out, loss_out,
                 idx_h, idx_t, idx_r, rows_h, rows_t, rows_r,
                 scores_v, lrow_v, sem):
        wid = lax.axis_index("s") * NC + lax.axis_index("c")
        pbase = wid * P
        nbase = half + wid * P

        for src, dst in ((h_hbm, idx_h), (t_hbm, idx_t), (r_hbm, idx_r)):
            for c in range(NCH // 2):
                pltpu.sync_copy(src.at[pl.ds(pbase + c * CH, CH)], dst.at[c])
            for c in range(NCH // 2):
                pltpu.sync_copy(src.at[pl.ds(nbase + c * CH, CH)],
                                dst.at[NCH // 2 + c])

        copies = []
        for tab, idx, dst in ((ent_hbm, idx_h, rows_h),
                              (ent_hbm, idx_t, rows_t),
                              (rel_hbm, idx_r, rows_r)):
            for c in range(NCH):
                copies.append(pltpu.async_copy(
                    tab.at[idx.at[c]], dst.at[pl.ds(c * CH, CH)], sem))
        for cp in copies:
            cp.wait()

        lane = lax.iota(jnp.int32, L)

        def group(g, lacc):
            prow = g * L + lane
            nrow = P + prow
            z = jnp.zeros((L,), jnp.float32)
            p_htr, p_h2, p_t2 = z, z, z
            n_htr, n_h2, n_t2 = z, z, z
            for d in range(D):
                col = jnp.full((L,), d, jnp.int32)
                ph = plsc.load_gather(rows_h, [prow, col])
                pt = plsc.load_gather(rows_t, [prow, col])
                pr = plsc.load_gather(rows_r, [prow, col])
                p_htr = p_htr + ph * pt * pr
                p_h2 = p_h2 + ph * ph
                p_t2 = p_t2 + pt * pt
                nh = plsc.load_gather(rows_h, [nrow, col])
                nt = plsc.load_gather(rows_t, [nrow, col])
                nr = plsc.load_gather(rows_r, [nrow, col])
                n_htr = n_htr + nh * nt * nr
                n_h2 = n_h2 + nh * nh
                n_t2 = n_t2 + nt * nt
            p_score = -(p_htr * _rsqrt_nr(jnp.maximum(p_h2, 1.0) *
                                          jnp.maximum(p_t2, 1.0)))
            n_score = -(n_htr * _rsqrt_nr(jnp.maximum(n_h2, 1.0) *
                                          jnp.maximum(n_t2, 1.0)))
            scores_v[pl.ds(g * L, L)] = p_score
            scores_v[pl.ds(P + g * L, L)] = n_score
            return lacc + jnp.maximum(p_score - n_score + margin, 0.0)

        lacc = lax.fori_loop(0, G, group, jnp.zeros((L,), jnp.float32))
        lrow_v[...] = jnp.full((L,), jnp.sum(lacc))
        pltpu.sync_copy(scores_v.at[pl.ds(0, P)], pos_out.at[pl.ds(pbase, P)])
        pltpu.sync_copy(scores_v.at[pl.ds(P, P)], neg_out.at[pl.ds(pbase, P)])
        pltpu.sync_copy(lrow_v, loss_out.at[wid])

    return distmult


def kernel(batch_h, batch_t, batch_r, batch_y, ent_emb, rel_emb):
    B = batch_h.shape[0]
    D = ent_emb.shape[1]
    fn = _build(B, D, 1.0)
    pos, neg, lpart = fn(batch_h.astype(jnp.int32), batch_t.astype(jnp.int32),
                         batch_r.astype(jnp.int32), ent_emb, rel_emb)
    loss = jnp.sum(lpart[:, 0])
    return (loss, pos, neg)


# speedup vs baseline: 1.0699x; 1.0699x over previous
import functools

import jax
import jax.numpy as jnp
import numpy as np
from jax import lax
from jax.experimental import pallas as pl
from jax.experimental.pallas import tpu as pltpu
from jax.experimental.pallas import tpu_sc as plsc

NC = 2
NS = 16
L = 16
CH = 128

_GATHER_DN = lax.GatherDimensionNumbers(
    offset_dims=(), collapsed_slice_dims=(0,), start_index_map=(0,))


def _perm(v, idx):
    return lax.gather(v, idx.reshape(L, 1), _GATHER_DN, slice_sizes=(1,),
                      mode=lax.GatherScatterMode.PROMISE_IN_BOUNDS)


def _rev4(j):
    return ((j & 1) << 3) | ((j & 2) << 1) | ((j & 4) >> 1) | ((j & 8) >> 3)


def _merge(left, right, xw):
    lane = lax.iota(jnp.int32, L)
    m = (lane & xw) == 0
    p = lane ^ xw
    out = []
    for a, b in zip(left, right):
        s = jnp.where(m, a, b)
        t = jnp.where(m, b, a)
        out.append(s + _perm(t, p))
    return tuple(out)


def _rsqrt_nr(m):
    i = plsc.bitcast(m, jnp.int32)
    seed = jnp.full((L,), 0x5F3759DF, jnp.int32) - lax.shift_right_logical(i, 1)
    y = plsc.bitcast(seed, jnp.float32)
    for _ in range(3):
        y = y * (1.5 - 0.5 * m * y * y)
    return y


@functools.lru_cache(maxsize=None)
def _build(B, D, margin):
    NW = NC * NS
    half = B // 2
    P = half // NW
    NCH = (2 * P) // CH
    G = P // L
    NCK = D // L

    mesh = plsc.VectorSubcoreMesh(
        core_axis_name="c", subcore_axis_name="s",
        num_cores=NC, num_subcores=NS)

    def row_accs(rows_h, rows_t, rows_r, row):
        htr = h2 = t2 = None
        for c in range(NCK):
            eh = rows_h[row, pl.ds(c * L, L)]
            et = rows_t[row, pl.ds(c * L, L)]
            er = rows_r[row, pl.ds(c * L, L)]
            ht = eh * et
            if c == 0:
                htr, h2, t2 = ht * er, eh * eh, et * et
            else:
                htr = htr + ht * er
                h2 = h2 + eh * eh
                t2 = t2 + et * et
        return (htr, h2, t2)

    def pack_group(rows_h, rows_t, rows_r, base_row):
        stack = []
        for j in range(L):
            lvl = 0
            node = row_accs(rows_h, rows_t, rows_r, base_row + _rev4(j))
            while stack and stack[-1][0] == lvl:
                _, left = stack.pop()
                node = _merge(left, node, 8 >> lvl)
                lvl += 1
            stack.append((lvl, node))
        htr, h2, t2 = stack[0][1]
        return -(htr * _rsqrt_nr(jnp.maximum(h2, 1.0) * jnp.maximum(t2, 1.0)))

    @functools.partial(
        pl.kernel,
        mesh=mesh,
        compiler_params=pltpu.CompilerParams(
            needs_layout_passes=False, use_tc_tiling_on_sc=False),
        out_type=(
            jax.ShapeDtypeStruct((half,), jnp.float32),
            jax.ShapeDtypeStruct((half,), jnp.float32),
            jax.ShapeDtypeStruct((NW, L), jnp.float32),
        ),
        scratch_types=[
            pltpu.VMEM((NCH, CH), jnp.int32),
            pltpu.VMEM((NCH, CH), jnp.int32),
            pltpu.VMEM((NCH, CH), jnp.int32),
            pltpu.VMEM((2 * P, D), jnp.float32),
            pltpu.VMEM((2 * P, D), jnp.float32),
            pltpu.VMEM((2 * P, D), jnp.float32),
            pltpu.VMEM((2 * P,), jnp.float32),
            pltpu.VMEM((L,), jnp.float32),
            pltpu.SemaphoreType.DMA,
        ],
    )
    def distmult(h_hbm, t_hbm, r_hbm, ent_hbm, rel_hbm,
                 pos_out, neg_out, loss_out,
                 idx_h, idx_t, idx_r, rows_h, rows_t, rows_r,
                 scores_v, lrow_v, sem):
        wid = lax.axis_index("s") * NC + lax.axis_index("c")
        pbase = wid * P
        nbase = half + wid * P

        for src, dst in ((h_hbm, idx_h), (t_hbm, idx_t), (r_hbm, idx_r)):
            for c in range(NCH // 2):
                pltpu.sync_copy(src.at[pl.ds(pbase + c * CH, CH)], dst.at[c])
            for c in range(NCH // 2):
                pltpu.sync_copy(src.at[pl.ds(nbase + c * CH, CH)],
                                dst.at[NCH // 2 + c])

        copies = []
        for tab, idx, dst in ((ent_hbm, idx_h, rows_h),
                              (ent_hbm, idx_t, rows_t),
                              (rel_hbm, idx_r, rows_r)):
            for c in range(NCH):
                copies.append(pltpu.async_copy(
                    tab.at[idx.at[c]], dst.at[pl.ds(c * CH, CH)], sem))
        for cp in copies:
            cp.wait()

        def group(g, lacc):
            p_score = pack_group(rows_h, rows_t, rows_r, g * L)
            n_score = pack_group(rows_h, rows_t, rows_r, P + g * L)
            scores_v[pl.ds(g * L, L)] = p_score
            scores_v[pl.ds(P + g * L, L)] = n_score
            return lacc + jnp.maximum(p_score - n_score + margin, 0.0)

        lacc = lax.fori_loop(0, G, group, jnp.zeros((L,), jnp.float32))
        lrow_v[...] = jnp.full((L,), jnp.sum(lacc))
        pltpu.sync_copy(scores_v.at[pl.ds(0, P)], pos_out.at[pl.ds(pbase, P)])
        pltpu.sync_copy(scores_v.at[pl.ds(P, P)], neg_out.at[pl.ds(pbase, P)])
        pltpu.sync_copy(lrow_v, loss_out.at[wid])

    return distmult


def kernel(batch_h, batch_t, batch_r, batch_y, ent_emb, rel_emb):
    B = batch_h.shape[0]
    D = ent_emb.shape[1]
    fn = _build(B, D, 1.0)
    pos, neg, lpart = fn(batch_h.astype(jnp.int32), batch_t.astype(jnp.int32),
                         batch_r.astype(jnp.int32), ent_emb, rel_emb)
    loss = jnp.sum(lpart[:, 0])
    return (loss, pos, neg)
